# baseline (speedup 1.0000x reference)
import functools

import jax
import jax.numpy as jnp
from jax import lax
from jax.experimental import pallas as pl
from jax.experimental.pallas import tpu as pltpu
from jax.experimental.pallas import tpu_sc as plsc

BATCH = 16384
FIELDS = 26
EMBED = 128
FIELD_STRIDE = 3847
TOTAL = BATCH * FIELDS

NUM_CORES = 2
NUM_SUBCORES = 16
NUM_WORKERS = NUM_CORES * NUM_SUBCORES
ROWS_PER_WORKER = TOTAL // NUM_WORKERS
GATHER_ROWS = 104
STEPS = ROWS_PER_WORKER // GATHER_ROWS
LANES = 16
K = 1
NSETS = 8
GROUPS = STEPS // K
SLACK = 6
OFF_ROWS = 2 * GATHER_ROWS
VECS_PER_OFF = OFF_ROWS // LANES


def _body(data_hbm, table_hbm, out_hbm, idx_v, rows_v, gsem, osem):
    wid = lax.axis_index("s") * NUM_CORES + lax.axis_index("c")
    base = wid * ROWS_PER_WORKER

    pltpu.sync_copy(data_hbm.at[wid], idx_v)

    def add_offsets(h, _):
        for v in range(VECS_PER_OFF):
            pos = h * OFF_ROWS + v * LANES + lax.iota(jnp.int32, LANES)
            off = lax.rem(pos, FIELDS) * FIELD_STRIDE
            sl = pl.ds(h * OFF_ROWS + v * LANES, LANES)
            idx_v[sl] = idx_v[sl] + off
        return 0

    lax.fori_loop(0, ROWS_PER_WORKER // OFF_ROWS, add_offsets, 0, unroll=2)

    def fire_gathers(t):
        s = lax.rem(t, NSETS)
        for k in range(K):
            g = t * K + k
            pltpu.async_copy(
                table_hbm.at[idx_v.at[pl.ds(g * GATHER_ROWS, GATHER_ROWS)]],
                rows_v.at[s * K + k],
                gsem,
            )

    def wait_gathers(t):
        s = lax.rem(t, NSETS)
        for k in range(K):
            g = t * K + k
            pltpu.make_async_copy(
                table_hbm.at[idx_v.at[pl.ds(g * GATHER_ROWS, GATHER_ROWS)]],
                rows_v.at[s * K + k],
                gsem,
            ).wait()

    def fire_stores(t):
        s = lax.rem(t, NSETS)
        for k in range(K):
            g = t * K + k
            pltpu.async_copy(
                rows_v.at[s * K + k],
                out_hbm.at[pl.ds(base + g * GATHER_ROWS, GATHER_ROWS)],
                osem,
            )

    def wait_stores(t):
        s = lax.rem(t, NSETS)
        for k in range(K):
            g = t * K + k
            pltpu.make_async_copy(
                rows_v.at[s * K + k],
                out_hbm.at[pl.ds(base + g * GATHER_ROWS, GATHER_ROWS)],
                osem,
            ).wait()

    fire_gathers(0)
    fire_gathers(1)

    def group_body(t, _):
        wait_gathers(t)
        fire_stores(t)

        @pl.when(t >= SLACK)
        def _():
            wait_stores(t - SLACK)

        @pl.when(t + 2 < GROUPS)
        def _():
            fire_gathers(t + 2)

        return 0

    lax.fori_loop(0, GROUPS, group_body, 0)
    for r in range(SLACK):
        wait_stores(GROUPS - SLACK + r)


@jax.jit
def _embed(data_flat, table):
    mesh = plsc.VectorSubcoreMesh(
        core_axis_name="c", subcore_axis_name="s",
        num_cores=NUM_CORES, num_subcores=NUM_SUBCORES,
    )
    run = functools.partial(
        pl.kernel,
        out_type=jax.ShapeDtypeStruct((TOTAL, EMBED), jnp.float32),
        mesh=mesh,
        scratch_types=[
            pltpu.VMEM((ROWS_PER_WORKER,), jnp.int32),
            pltpu.VMEM((NSETS * K, GATHER_ROWS, EMBED), jnp.float32),
            pltpu.SemaphoreType.DMA,
            pltpu.SemaphoreType.DMA,
        ],
    )(_body)
    return run(data_flat, table)


def kernel(data, table):
    data_flat = data.astype(jnp.int32).reshape(NUM_WORKERS, ROWS_PER_WORKER)
    out = _embed(data_flat, table)
    return out.reshape(BATCH, FIELDS, EMBED)


# speedup vs baseline: 1.0061x; 1.0061x over previous
import functools

import jax
import jax.numpy as jnp
from jax import lax
from jax.experimental import pallas as pl
from jax.experimental.pallas import tpu as pltpu
from jax.experimental.pallas import tpu_sc as plsc

BATCH = 16384
FIELDS = 26
EMBED = 128
FIELD_STRIDE = 3847
TOTAL = BATCH * FIELDS

NUM_CORES = 2
NUM_SUBCORES = 16
NUM_WORKERS = NUM_CORES * NUM_SUBCORES
ROWS_PER_WORKER = TOTAL // NUM_WORKERS
GATHER_ROWS = 104
STEPS = ROWS_PER_WORKER // GATHER_ROWS
LANES = 16
K = 2
NSETS = 4
GROUPS = STEPS // K
GROUP_ROWS = K * GATHER_ROWS
VECS_PER_GROUP = GROUP_ROWS // LANES


def _body(data_hbm, table_hbm, out_hbm, idx_v, rows_v, gsem, osem):
    wid = lax.axis_index("s") * NUM_CORES + lax.axis_index("c")
    base = wid * ROWS_PER_WORKER

    pltpu.sync_copy(data_hbm.at[wid], idx_v)

    def add_offsets(t):
        for v in range(VECS_PER_GROUP):
            pos = t * GROUP_ROWS + v * LANES + lax.iota(jnp.int32, LANES)
            off = lax.rem(pos, FIELDS) * FIELD_STRIDE
            sl = pl.ds(t * GROUP_ROWS + v * LANES, LANES)
            idx_v[sl] = idx_v[sl] + off

    def fire_gathers(t):
        s = lax.rem(t, NSETS)
        for k in range(K):
            g = t * K + k
            pltpu.async_copy(
                table_hbm.at[idx_v.at[pl.ds(g * GATHER_ROWS, GATHER_ROWS)]],
                rows_v.at[s * K + k],
                gsem,
            )

    def wait_gathers(t):
        s = lax.rem(t, NSETS)
        for k in range(K):
            g = t * K + k
            pltpu.make_async_copy(
                table_hbm.at[idx_v.at[pl.ds(g * GATHER_ROWS, GATHER_ROWS)]],
                rows_v.at[s * K + k],
                gsem,
            ).wait()

    def fire_stores(t):
        s = lax.rem(t, NSETS)
        for k in range(K):
            g = t * K + k
            pltpu.async_copy(
                rows_v.at[s * K + k],
                out_hbm.at[pl.ds(base + g * GATHER_ROWS, GATHER_ROWS)],
                osem,
            )

    def wait_stores(t):
        s = lax.rem(t, NSETS)
        for k in range(K):
            g = t * K + k
            pltpu.make_async_copy(
                rows_v.at[s * K + k],
                out_hbm.at[pl.ds(base + g * GATHER_ROWS, GATHER_ROWS)],
                osem,
            ).wait()

    add_offsets(0)
    fire_gathers(0)
    add_offsets(1)
    fire_gathers(1)

    def group_body(t, _):
        @pl.when(t + 2 < GROUPS)
        def _():
            add_offsets(t + 2)

        wait_gathers(t)
        fire_stores(t)

        @pl.when(t >= 2)
        def _():
            wait_stores(t - 2)

        @pl.when(t + 2 < GROUPS)
        def _():
            fire_gathers(t + 2)

        return 0

    lax.fori_loop(0, GROUPS, group_body, 0)
    wait_stores(GROUPS - 2)
    wait_stores(GROUPS - 1)


@jax.jit
def _embed(data_flat, table):
    mesh = plsc.VectorSubcoreMesh(
        core_axis_name="c", subcore_axis_name="s",
        num_cores=NUM_CORES, num_subcores=NUM_SUBCORES,
    )
    run = functools.partial(
        pl.kernel,
        out_type=jax.ShapeDtypeStruct((TOTAL, EMBED), jnp.float32),
        mesh=mesh,
        scratch_types=[
            pltpu.VMEM((ROWS_PER_WORKER,), jnp.int32),
            pltpu.VMEM((NSETS * K, GATHER_ROWS, EMBED), jnp.float32),
            pltpu.SemaphoreType.DMA,
            pltpu.SemaphoreType.DMA,
        ],
    )(_body)
    return run(data_flat, table)


def kernel(data, table):
    data_flat = data.astype(jnp.int32).reshape(NUM_WORKERS, ROWS_PER_WORKER)
    out = _embed(data_flat, table)
    return out.reshape(BATCH, FIELDS, EMBED)
